# baseline (speedup 1.0000x reference)
import dataclasses
import functools
import math

import jax
import jax.numpy as jnp
from jax import lax
from jax.experimental import pallas as pl
from jax.experimental.pallas import tpu as pltpu
from jax.experimental.pallas import tpu_sc as plsc

N = 4096
C = 256
TDIM = 512
E = 131072
K = 2048

_INTERPRET = jax.default_backend() == "cpu"

_PREC = jax.lax.Precision.DEFAULT

E_AUG = E + K
EPT = E_AUG // 16
NB = EPT // 128
CH = 128
NCH = K // CH
ZW = 16384


def _build_kernel(src_hbm, dst_hbm, w_hbm, rank_hbm, perm_hbm, x_hbm,
                  p_hbm, qt_hbm, xg_hbm,
                  a_v, c_v, w_v, rank_v, idx3, r3, upd3, zeros_v,
                  permv, rows_v, shared, sem):
    cid = lax.axis_index("c")
    sid = lax.axis_index("s")
    base = sid * EPT

    wid = sid * 2 + cid
    pltpu.sync_copy(perm_hbm.at[pl.ds(wid * (K // 32), K // 32)], permv)
    pltpu.async_copy(x_hbm.at[permv], rows_v, sem).wait()
    pltpu.sync_copy(rows_v, xg_hbm.at[pl.ds(wid * (K // 32), K // 32)])

    @pl.when(cid == 0)
    def _():
        pltpu.sync_copy(src_hbm.at[pl.ds(base, EPT)], a_v)
        pltpu.sync_copy(dst_hbm.at[pl.ds(base, EPT)], c_v)

    @pl.when(cid != 0)
    def _():
        pltpu.sync_copy(dst_hbm.at[pl.ds(base, EPT)], a_v)
        pltpu.sync_copy(src_hbm.at[pl.ds(base, EPT)], c_v)

    pltpu.sync_copy(w_hbm.at[pl.ds(base, EPT)], w_v)
    pltpu.sync_copy(rank_hbm, rank_v)

    @pl.loop(0, ZW // 16)
    def _(q):
        zeros_v[pl.ds(q * 16, 16)] = jnp.zeros((16,), jnp.float32)

    @pl.loop(0, NB)
    def _(j):
        @pl.loop(0, 8)
        def _(t):
            a16 = a_v[pl.ds(j * 128 + t * 16, 16)]
            c16 = c_v[pl.ds(j * 128 + t * 16, 16)]
            r16 = plsc.load_gather(rank_v, [a16])
            idx3[j, pl.ds(t * 16, 16)] = (r16 & (CH - 1)) * N + c16
            r3[j, pl.ds(t * 16, 16)] = r16

    part = CH * N // 16

    @pl.loop(0, NCH)
    def _(k):
        @pl.loop(0, part // ZW)
        def _(q):
            pltpu.async_copy(zeros_v, shared.at[pl.ds(sid * part + q * ZW, ZW)],
                             sem)

        @pl.loop(0, part // ZW)
        def _(q):
            pltpu.make_async_copy(
                zeros_v, shared.at[pl.ds(sid * part + q * ZW, ZW)], sem).wait()
        plsc.subcore_barrier()
        lo = k * CH

        @pl.loop(0, NB)
        def _(j):
            @pl.loop(0, 8)
            def _(t):
                r16 = r3[j, pl.ds(t * 16, 16)]
                w16 = w_v[pl.ds(j * 128 + t * 16, 16)]
                m = jnp.logical_and(r16 >= lo, r16 < lo + CH)
                upd3[j, pl.ds(t * 16, 16)] = jnp.where(m, w16, 0.0)

        @pl.loop(0, NB)
        def _(j):
            pltpu.async_copy(upd3.at[j], shared.at[idx3.at[j]], sem, add=True)

        @pl.loop(0, NB)
        def _(j):
            pltpu.make_async_copy(upd3.at[j], shared.at[idx3.at[j]], sem).wait()

        plsc.subcore_barrier()

        rows_per_tile = CH // 16

        @pl.when(cid == 0)
        def _():
            @pl.loop(0, rows_per_tile)
            def _(r):
                rl = sid * rows_per_tile + r
                pltpu.async_copy(shared.at[pl.ds(rl * N, N)],
                                 p_hbm.at[k * CH + rl], sem)

            @pl.loop(0, rows_per_tile)
            def _(r):
                rl = sid * rows_per_tile + r
                pltpu.make_async_copy(shared.at[pl.ds(rl * N, N)],
                                      p_hbm.at[k * CH + rl], sem).wait()

        @pl.when(cid != 0)
        def _():
            @pl.loop(0, rows_per_tile)
            def _(r):
                rl = sid * rows_per_tile + r
                pltpu.async_copy(shared.at[pl.ds(rl * N, N)],
                                 qt_hbm.at[k * CH + rl], sem)

            @pl.loop(0, rows_per_tile)
            def _(r):
                rl = sid * rows_per_tile + r
                pltpu.make_async_copy(shared.at[pl.ds(rl * N, N)],
                                      qt_hbm.at[k * CH + rl], sem).wait()


def _build_pq(src_aug, dst_aug, w_aug, rank_flat, perm, x):
    mesh = plsc.VectorSubcoreMesh(core_axis_name="c", subcore_axis_name="s")
    cp = pltpu.CompilerParams()
    if "needs_layout_passes" in pltpu.CompilerParams.__dataclass_fields__:
        cp = dataclasses.replace(cp, needs_layout_passes=False)
    f = pl.kernel(
        _build_kernel,
        compiler_params=cp,
        out_type=[jax.ShapeDtypeStruct((K, N), jnp.float32),
                  jax.ShapeDtypeStruct((K, N), jnp.float32),
                  jax.ShapeDtypeStruct((K, C), jnp.float32)],
        mesh=mesh,
        scratch_types=[
            pltpu.VMEM((EPT,), jnp.int32),
            pltpu.VMEM((EPT,), jnp.int32),
            pltpu.VMEM((EPT,), jnp.float32),
            pltpu.VMEM((N,), jnp.int32),
            pltpu.VMEM((NB, 128), jnp.int32),
            pltpu.VMEM((NB, 128), jnp.int32),
            pltpu.VMEM((NB, 128), jnp.float32),
            pltpu.VMEM((ZW,), jnp.float32),
            pltpu.VMEM((K // 32,), jnp.int32),
            pltpu.VMEM((K // 32, C), jnp.float32),
            pltpu.VMEM_SHARED((CH * N,), jnp.float32),
            pltpu.SemaphoreType.DMA,
        ])
    return f(src_aug, dst_aug, w_aug, rank_flat, perm, x)


def _rank_kernel(scol_ref, srow_ref, rank_ref):
    i = pl.program_id(0)
    bm = scol_ref.shape[0]
    sc = scol_ref[...]
    sr = srow_ref[...]
    gt = (sr > sc).astype(jnp.int32)
    jrow = jax.lax.broadcasted_iota(jnp.int32, (bm, N), 1)
    gidx = i * bm + jax.lax.broadcasted_iota(jnp.int32, (bm, N), 0)
    tie = jnp.logical_and(sr == sc, jrow < gidx).astype(jnp.int32)
    rank_ref[...] = jnp.sum(gt + tie, axis=1, keepdims=True)


def _perm_kernel(rrow_ref, srow_ref, perm_ref, sp_ref):
    i = pl.program_id(0)
    bm = perm_ref.shape[0]
    rr = rrow_ref[...]
    sr = srow_ref[...]
    p = i * bm + jax.lax.broadcasted_iota(jnp.int32, (bm, N), 0)
    hit = (rr == p)
    jrow = jax.lax.broadcasted_iota(jnp.int32, (bm, N), 1)
    perm_ref[...] = jnp.sum(jnp.where(hit, jrow, 0), axis=1, keepdims=True)
    sp_ref[...] = jnp.sum(jnp.where(hit, sr, 0.0), axis=1, keepdims=True)


def _ap_kernel(p_ref, qt_ref, ap_ref, colsum_ref, qtb_ref, *, bm, bn):
    j = pl.program_id(0)
    i = pl.program_id(1)

    @pl.when(i == 0)
    def _():
        qtb_ref[...] = qt_ref[...].astype(jnp.bfloat16)

    acc = jax.lax.dot_general(
        p_ref[...].astype(jnp.bfloat16), qtb_ref[...],
        (((1,), (1,)), ((), ())),
        preferred_element_type=jnp.float32,
        precision=_PREC)
    gi = i * bm + jax.lax.broadcasted_iota(jnp.int32, (bm, bn), 0)
    gj = j * bn + jax.lax.broadcasted_iota(jnp.int32, (bm, bn), 1)
    acc = jnp.where(gi == gj, 0.0, acc)
    ap_ref[...] = acc
    part = jnp.sum(acc, axis=0, keepdims=True)[None]

    @pl.when(i == 0)
    def _init():
        colsum_ref[...] = part

    @pl.when(i != 0)
    def _acc():
        colsum_ref[...] += part


def kernel(x, edge_index, edge_weight, t, W1, b1, W2, b2,
           ln1_g, ln1_b, ln2_g, ln2_b, pool_w, time_W, time_b):
    src, dst = edge_index[0], edge_index[1]

    score = (x * pool_w).sum(-1) / jnp.linalg.norm(pool_w)
    score = jnp.tanh(score)

    srow = score.reshape(1, N)
    scol = score.reshape(N, 1)

    bm = 512
    rank = pl.pallas_call(
        _rank_kernel,
        grid=(N // bm,),
        in_specs=[pl.BlockSpec((bm, 1), lambda i: (i, 0)),
                  pl.BlockSpec((1, N), lambda i: (0, 0))],
        out_specs=pl.BlockSpec((bm, 1), lambda i: (i, 0)),
        out_shape=jax.ShapeDtypeStruct((N, 1), jnp.int32),
        interpret=_INTERPRET,
    )(scol, srow)

    rrow = rank.reshape(1, N)
    perm_full, sp_full = pl.pallas_call(
        _perm_kernel,
        grid=(N // bm,),
        in_specs=[pl.BlockSpec((1, N), lambda i: (0, 0)),
                  pl.BlockSpec((1, N), lambda i: (0, 0))],
        out_specs=[pl.BlockSpec((bm, 1), lambda i: (i, 0)),
                   pl.BlockSpec((bm, 1), lambda i: (i, 0))],
        out_shape=[jax.ShapeDtypeStruct((N, 1), jnp.int32),
                   jax.ShapeDtypeStruct((N, 1), jnp.float32)],
        interpret=_INTERPRET,
    )(rrow, srow)
    perm = perm_full[:K, 0]
    sp = sp_full[:K]

    rk = rank[:, 0]
    w0 = jnp.where(src == dst, 0.0, edge_weight)
    src_aug = jnp.concatenate([src, perm])
    dst_aug = jnp.concatenate([dst, perm])
    w_aug = jnp.concatenate([w0, jnp.ones((K,), jnp.float32)])
    P, QT, xg = _build_pq(src_aug, dst_aug, w_aug, rk, perm, x)

    bm3, bn3 = 128, 1024
    ap, colsum = pl.pallas_call(
        functools.partial(_ap_kernel, bm=bm3, bn=bn3),
        grid=(K // bn3, K // bm3),
        in_specs=[pl.BlockSpec((bm3, N), lambda j, i: (i, 0)),
                  pl.BlockSpec((bn3, N), lambda j, i: (j, 0))],
        out_specs=[pl.BlockSpec((bm3, bn3), lambda j, i: (i, j)),
                   pl.BlockSpec((1, 1, bn3), lambda j, i: (j, 0, 0))],
        out_shape=[jax.ShapeDtypeStruct((K, K), jnp.float32),
                   jax.ShapeDtypeStruct((K // bn3, 1, bn3), jnp.float32)],
        compiler_params=pltpu.CompilerParams(
            dimension_semantics=("parallel", "arbitrary")),
        scratch_shapes=[pltpu.VMEM((bn3, N), jnp.bfloat16)],
        interpret=_INTERPRET,
    )(P, QT)
    colsum = colsum.reshape(K, 1)

    bm4 = 512
    t_row = t.reshape(1, TDIM)
    tb_row = time_b.reshape(1, C)

    def gcn_layer(xin, spv, W, b, lng, lnb, with_time):
        body = functools.partial(_gcn2_kernel, bm=bm4, with_time=with_time,
                                 with_sp=spv is not None)
        ins = [ap, colsum, xin]
        specs = [pl.BlockSpec((K, bm4), lambda i: (0, i)),
                 pl.BlockSpec((K, 1), lambda i: (0, 0)),
                 pl.BlockSpec((K, C), lambda i: (0, 0))]
        if spv is not None:
            ins.append(spv)
            specs.append(pl.BlockSpec((K, 1), lambda i: (0, 0)))
        ins += [colsum, xin]
        specs += [pl.BlockSpec((bm4, 1), lambda i: (i, 0)),
                  pl.BlockSpec((bm4, C), lambda i: (i, 0))]
        if spv is not None:
            ins.append(spv)
            specs.append(pl.BlockSpec((bm4, 1), lambda i: (i, 0)))
        ins += [W, b.reshape(1, C), lng.reshape(1, C), lnb.reshape(1, C)]
        specs += [pl.BlockSpec((C, C), lambda i: (0, 0)),
                  pl.BlockSpec((1, C), lambda i: (0, 0)),
                  pl.BlockSpec((1, C), lambda i: (0, 0)),
                  pl.BlockSpec((1, C), lambda i: (0, 0))]
        if with_time:
            ins += [time_W, t_row, tb_row]
            specs += [pl.BlockSpec((C, TDIM), lambda i: (0, 0)),
                      pl.BlockSpec((1, TDIM), lambda i: (0, 0)),
                      pl.BlockSpec((1, C), lambda i: (0, 0))]
        return pl.pallas_call(
            body,
            grid=(K // bm4,),
            in_specs=specs,
            out_specs=pl.BlockSpec((bm4, C), lambda i: (i, 0)),
            out_shape=jax.ShapeDtypeStruct((K, C), jnp.float32),
            compiler_params=pltpu.CompilerParams(
                dimension_semantics=("parallel",)),
            interpret=_INTERPRET,
        )(*ins)

    h1 = gcn_layer(xg, sp, W1, b1, ln1_g, ln1_b, True)
    h = gcn_layer(h1, None, W2, b2, ln2_g, ln2_b, False)
    return h, ap, perm


def _gcn2_kernel(*refs, bm, with_time, with_sp):
    if with_time:
        *refs, tw_ref, t_ref, tb_ref, out_ref = refs
        refs = refs + [out_ref]
    if with_sp:
        (ap_ref, colsum_ref, xin_ref, sp_ref, cs_blk_ref, xin_blk_ref,
         sp_blk_ref, w_ref, b_ref, lng_ref, lnb_ref, out_ref) = refs
    else:
        (ap_ref, colsum_ref, xin_ref, cs_blk_ref, xin_blk_ref,
         w_ref, b_ref, lng_ref, lnb_ref, out_ref) = refs
        sp_ref = sp_blk_ref = None
    dinv = jax.lax.rsqrt(colsum_ref[...] + 2.0)
    xin = xin_ref[...]
    xin_blk = xin_blk_ref[...]
    if sp_ref is not None:
        xin = xin * sp_ref[...]
        xin_blk = xin_blk * sp_blk_ref[...]
    z = jax.lax.dot_general(
        xin * dinv, w_ref[...], (((1,), (1,)), ((), ())),
        preferred_element_type=jnp.float32,
        precision=_PREC)
    y = jax.lax.dot_general(
        ap_ref[...], z, (((0,), (0,)), ((), ())),
        preferred_element_type=jnp.float32,
        precision=_PREC)
    dblk = jax.lax.rsqrt(cs_blk_ref[...] + 2.0)
    zblk = jax.lax.dot_general(
        xin_blk * dblk, w_ref[...], (((1,), (1,)), ((), ())),
        preferred_element_type=jnp.float32,
        precision=_PREC)
    g = dblk * (y + 2.0 * zblk) + b_ref[...]
    g = jnp.maximum(g, 0.0)
    mu = jnp.mean(g, axis=1, keepdims=True)
    var = jnp.mean((g - mu) ** 2, axis=1, keepdims=True)
    h = (g - mu) * jax.lax.rsqrt(var + 1e-5) * lng_ref[...] + lnb_ref[...]
    if with_time:
        tv = jax.lax.dot_general(
            t_ref[...], tw_ref[...], (((1,), (1,)), ((), ())),
            preferred_element_type=jnp.float32,
            precision=_PREC)
        h = h + jnp.maximum(tv + tb_ref[...], 0.0)
    out_ref[...] = h


# speedup vs baseline: 1.2225x; 1.2225x over previous
import dataclasses
import functools
import math

import jax
import jax.numpy as jnp
from jax import lax
from jax.experimental import pallas as pl
from jax.experimental.pallas import tpu as pltpu
from jax.experimental.pallas import tpu_sc as plsc

N = 4096
C = 256
TDIM = 512
E = 131072
K = 2048

_PREC = jax.lax.Precision.DEFAULT

E_AUG = E + K
EPT = E_AUG // 16
NB = EPT // 128
CH = 256
NCH = K // CH
ZW = 4096
GB = 16


def _build_kernel(src_hbm, dst_hbm, w_hbm, rank_hbm, perm_hbm, x_hbm,
                  p_hbm, qt_hbm, xg_hbm,
                  a_v, c_v, w_v, rank_v, idx3, r3, upd3, zeros_v,
                  permv, rows_v, shared, sem):
    cid = lax.axis_index("c")
    sid = lax.axis_index("s")
    base = sid * EPT

    wid = sid * 2 + cid
    pltpu.sync_copy(perm_hbm.at[pl.ds(wid * (K // 32), K // 32)], permv)

    @pl.loop(0, (K // 32) // GB)
    def _(g):
        pltpu.async_copy(x_hbm.at[permv.at[pl.ds(g * GB, GB)]], rows_v,
                         sem).wait()
        pltpu.sync_copy(rows_v, xg_hbm.at[pl.ds(wid * (K // 32) + g * GB, GB)])

    @pl.when(cid == 0)
    def _():
        pltpu.sync_copy(src_hbm.at[pl.ds(base, EPT)], a_v)
        pltpu.sync_copy(dst_hbm.at[pl.ds(base, EPT)], c_v)

    @pl.when(cid != 0)
    def _():
        pltpu.sync_copy(dst_hbm.at[pl.ds(base, EPT)], a_v)
        pltpu.sync_copy(src_hbm.at[pl.ds(base, EPT)], c_v)

    pltpu.sync_copy(w_hbm.at[pl.ds(base, EPT)], w_v)
    pltpu.sync_copy(rank_hbm, rank_v)

    @pl.loop(0, ZW // 16)
    def _(q):
        zeros_v[pl.ds(q * 16, 16)] = jnp.zeros((16,), jnp.float32)

    @pl.loop(0, NB)
    def _(j):
        @pl.loop(0, 8)
        def _(t):
            a16 = a_v[pl.ds(j * 128 + t * 16, 16)]
            c16 = c_v[pl.ds(j * 128 + t * 16, 16)]
            r16 = plsc.load_gather(rank_v, [a16])
            idx3[j, pl.ds(t * 16, 16)] = (r16 & (CH - 1)) * N + c16
            r3[j, pl.ds(t * 16, 16)] = r16

    part = CH * N // 16

    @pl.loop(0, NCH)
    def _(k):
        @pl.loop(0, part // ZW)
        def _(q):
            pltpu.async_copy(zeros_v, shared.at[pl.ds(sid * part + q * ZW, ZW)],
                             sem)

        @pl.loop(0, part // ZW)
        def _(q):
            pltpu.make_async_copy(
                zeros_v, shared.at[pl.ds(sid * part + q * ZW, ZW)], sem).wait()
        plsc.subcore_barrier()
        lo = k * CH

        @pl.loop(0, NB)
        def _(j):
            @pl.loop(0, 8)
            def _(t):
                r16 = r3[j, pl.ds(t * 16, 16)]
                w16 = w_v[pl.ds(j * 128 + t * 16, 16)]
                m = jnp.logical_and(r16 >= lo, r16 < lo + CH)
                upd3[j, pl.ds(t * 16, 16)] = jnp.where(m, w16, 0.0)

        @pl.loop(0, NB)
        def _(j):
            pltpu.async_copy(upd3.at[j], shared.at[idx3.at[j]], sem, add=True)

        @pl.loop(0, NB)
        def _(j):
            pltpu.make_async_copy(upd3.at[j], shared.at[idx3.at[j]], sem).wait()

        plsc.subcore_barrier()

        rows_per_tile = CH // 16

        @pl.when(cid == 0)
        def _():
            @pl.loop(0, rows_per_tile)
            def _(r):
                rl = sid * rows_per_tile + r
                pltpu.async_copy(shared.at[pl.ds(rl * N, N)],
                                 p_hbm.at[k * CH + rl], sem)

            @pl.loop(0, rows_per_tile)
            def _(r):
                rl = sid * rows_per_tile + r
                pltpu.make_async_copy(shared.at[pl.ds(rl * N, N)],
                                      p_hbm.at[k * CH + rl], sem).wait()

        @pl.when(cid != 0)
        def _():
            @pl.loop(0, rows_per_tile)
            def _(r):
                rl = sid * rows_per_tile + r
                pltpu.async_copy(shared.at[pl.ds(rl * N, N)],
                                 qt_hbm.at[k * CH + rl], sem)

            @pl.loop(0, rows_per_tile)
            def _(r):
                rl = sid * rows_per_tile + r
                pltpu.make_async_copy(shared.at[pl.ds(rl * N, N)],
                                      qt_hbm.at[k * CH + rl], sem).wait()


def _build_pq(src_aug, dst_aug, w_aug, rank_flat, perm, x):
    mesh = plsc.VectorSubcoreMesh(core_axis_name="c", subcore_axis_name="s")
    cp = pltpu.CompilerParams()
    if "needs_layout_passes" in pltpu.CompilerParams.__dataclass_fields__:
        cp = dataclasses.replace(cp, needs_layout_passes=False)
    f = pl.kernel(
        _build_kernel,
        compiler_params=cp,
        out_type=[jax.ShapeDtypeStruct((K, N), jnp.float32),
                  jax.ShapeDtypeStruct((K, N), jnp.float32),
                  jax.ShapeDtypeStruct((K, C), jnp.float32)],
        mesh=mesh,
        scratch_types=[
            pltpu.VMEM((EPT,), jnp.int32),
            pltpu.VMEM((EPT,), jnp.int32),
            pltpu.VMEM((EPT,), jnp.float32),
            pltpu.VMEM((N,), jnp.int32),
            pltpu.VMEM((NB, 128), jnp.int32),
            pltpu.VMEM((NB, 128), jnp.int32),
            pltpu.VMEM((NB, 128), jnp.float32),
            pltpu.VMEM((ZW,), jnp.float32),
            pltpu.VMEM((K // 32,), jnp.int32),
            pltpu.VMEM((GB, C), jnp.float32),
            pltpu.VMEM_SHARED((CH * N,), jnp.float32),
            pltpu.SemaphoreType.DMA,
        ])
    return f(src_aug, dst_aug, w_aug, rank_flat, perm, x)


def _rank_kernel(scol_ref, srow_ref, rank_ref):
    i = pl.program_id(0)
    bm = scol_ref.shape[0]
    sc = scol_ref[...]
    sr = srow_ref[...]
    gt = (sr > sc).astype(jnp.int32)
    jrow = jax.lax.broadcasted_iota(jnp.int32, (bm, N), 1)
    gidx = i * bm + jax.lax.broadcasted_iota(jnp.int32, (bm, N), 0)
    tie = jnp.logical_and(sr == sc, jrow < gidx).astype(jnp.int32)
    rank_ref[...] = jnp.sum(gt + tie, axis=1, keepdims=True)


def _perm_kernel(rrow_ref, srow_ref, perm_ref, sp_ref):
    i = pl.program_id(0)
    bm = perm_ref.shape[0]
    rr = rrow_ref[...]
    sr = srow_ref[...]
    p = i * bm + jax.lax.broadcasted_iota(jnp.int32, (bm, N), 0)
    hit = (rr == p)
    jrow = jax.lax.broadcasted_iota(jnp.int32, (bm, N), 1)
    perm_ref[...] = jnp.sum(jnp.where(hit, jrow, 0), axis=1, keepdims=True)
    sp_ref[...] = jnp.sum(jnp.where(hit, sr, 0.0), axis=1, keepdims=True)


def _ap_kernel(p_ref, qt_ref, ap_ref, colsum_ref, qtb_ref, *, bm, bn):
    j = pl.program_id(0)
    i = pl.program_id(1)

    @pl.when(i == 0)
    def _():
        qtb_ref[...] = qt_ref[...].astype(jnp.bfloat16)

    acc = jax.lax.dot_general(
        p_ref[...].astype(jnp.bfloat16), qtb_ref[...],
        (((1,), (1,)), ((), ())),
        preferred_element_type=jnp.float32,
        precision=_PREC)
    gi = i * bm + jax.lax.broadcasted_iota(jnp.int32, (bm, bn), 0)
    gj = j * bn + jax.lax.broadcasted_iota(jnp.int32, (bm, bn), 1)
    acc = jnp.where(gi == gj, 0.0, acc)
    ap_ref[...] = acc
    part = jnp.sum(acc, axis=0, keepdims=True)[None]

    @pl.when(i == 0)
    def _init():
        colsum_ref[...] = part

    @pl.when(i != 0)
    def _acc():
        colsum_ref[...] += part


def kernel(x, edge_index, edge_weight, t, W1, b1, W2, b2,
           ln1_g, ln1_b, ln2_g, ln2_b, pool_w, time_W, time_b):
    src, dst = edge_index[0], edge_index[1]

    score = (x * pool_w).sum(-1) / jnp.linalg.norm(pool_w)
    score = jnp.tanh(score)

    srow = score.reshape(1, N)
    scol = score.reshape(N, 1)

    bm = 512
    rank = pl.pallas_call(
        _rank_kernel,
        grid=(N // bm,),
        in_specs=[pl.BlockSpec((bm, 1), lambda i: (i, 0)),
                  pl.BlockSpec((1, N), lambda i: (0, 0))],
        out_specs=pl.BlockSpec((bm, 1), lambda i: (i, 0)),
        out_shape=jax.ShapeDtypeStruct((N, 1), jnp.int32),
    )(scol, srow)

    rrow = rank.reshape(1, N)
    perm_full, sp_full = pl.pallas_call(
        _perm_kernel,
        grid=(N // bm,),
        in_specs=[pl.BlockSpec((1, N), lambda i: (0, 0)),
                  pl.BlockSpec((1, N), lambda i: (0, 0))],
        out_specs=[pl.BlockSpec((bm, 1), lambda i: (i, 0)),
                   pl.BlockSpec((bm, 1), lambda i: (i, 0))],
        out_shape=[jax.ShapeDtypeStruct((N, 1), jnp.int32),
                   jax.ShapeDtypeStruct((N, 1), jnp.float32)],
    )(rrow, srow)
    perm = perm_full[:K, 0]
    sp = sp_full[:K]

    rk = rank[:, 0]
    w0 = jnp.where(src == dst, 0.0, edge_weight)
    src_aug = jnp.concatenate([src, perm])
    dst_aug = jnp.concatenate([dst, perm])
    w_aug = jnp.concatenate([w0, jnp.ones((K,), jnp.float32)])
    P, QT, xg = _build_pq(src_aug, dst_aug, w_aug, rk, perm, x)

    bm3, bn3 = 128, 1024
    ap, colsum = pl.pallas_call(
        functools.partial(_ap_kernel, bm=bm3, bn=bn3),
        grid=(K // bn3, K // bm3),
        in_specs=[pl.BlockSpec((bm3, N), lambda j, i: (i, 0)),
                  pl.BlockSpec((bn3, N), lambda j, i: (j, 0))],
        out_specs=[pl.BlockSpec((bm3, bn3), lambda j, i: (i, j)),
                   pl.BlockSpec((1, 1, bn3), lambda j, i: (j, 0, 0))],
        out_shape=[jax.ShapeDtypeStruct((K, K), jnp.float32),
                   jax.ShapeDtypeStruct((K // bn3, 1, bn3), jnp.float32)],
        compiler_params=pltpu.CompilerParams(
            dimension_semantics=("parallel", "arbitrary")),
        scratch_shapes=[pltpu.VMEM((bn3, N), jnp.bfloat16)],
    )(P, QT)
    colsum = colsum.reshape(K, 1)

    bm4 = 512
    t_row = t.reshape(1, TDIM)
    tb_row = time_b.reshape(1, C)

    def gcn_layer(xin, spv, W, b, lng, lnb, with_time):
        body = functools.partial(_gcn2_kernel, bm=bm4, with_time=with_time,
                                 with_sp=spv is not None)
        ins = [ap, colsum, xin]
        specs = [pl.BlockSpec((K, bm4), lambda i: (0, i)),
                 pl.BlockSpec((K, 1), lambda i: (0, 0)),
                 pl.BlockSpec((K, C), lambda i: (0, 0))]
        if spv is not None:
            ins.append(spv)
            specs.append(pl.BlockSpec((K, 1), lambda i: (0, 0)))
        ins += [colsum, xin]
        specs += [pl.BlockSpec((bm4, 1), lambda i: (i, 0)),
                  pl.BlockSpec((bm4, C), lambda i: (i, 0))]
        if spv is not None:
            ins.append(spv)
            specs.append(pl.BlockSpec((bm4, 1), lambda i: (i, 0)))
        ins += [W, b.reshape(1, C), lng.reshape(1, C), lnb.reshape(1, C)]
        specs += [pl.BlockSpec((C, C), lambda i: (0, 0)),
                  pl.BlockSpec((1, C), lambda i: (0, 0)),
                  pl.BlockSpec((1, C), lambda i: (0, 0)),
                  pl.BlockSpec((1, C), lambda i: (0, 0))]
        if with_time:
            ins += [time_W, t_row, tb_row]
            specs += [pl.BlockSpec((C, TDIM), lambda i: (0, 0)),
                      pl.BlockSpec((1, TDIM), lambda i: (0, 0)),
                      pl.BlockSpec((1, C), lambda i: (0, 0))]
        return pl.pallas_call(
            body,
            grid=(K // bm4,),
            in_specs=specs,
            out_specs=pl.BlockSpec((bm4, C), lambda i: (i, 0)),
            out_shape=jax.ShapeDtypeStruct((K, C), jnp.float32),
            compiler_params=pltpu.CompilerParams(
                dimension_semantics=("parallel",)),
            )(*ins)

    h1 = gcn_layer(xg, sp, W1, b1, ln1_g, ln1_b, True)
    h = gcn_layer(h1, None, W2, b2, ln2_g, ln2_b, False)
    return h, ap, perm


def _gcn2_kernel(*refs, bm, with_time, with_sp):
    if with_time:
        *refs, tw_ref, t_ref, tb_ref, out_ref = refs
        refs = refs + [out_ref]
    if with_sp:
        (ap_ref, colsum_ref, xin_ref, sp_ref, cs_blk_ref, xin_blk_ref,
         sp_blk_ref, w_ref, b_ref, lng_ref, lnb_ref, out_ref) = refs
    else:
        (ap_ref, colsum_ref, xin_ref, cs_blk_ref, xin_blk_ref,
         w_ref, b_ref, lng_ref, lnb_ref, out_ref) = refs
        sp_ref = sp_blk_ref = None
    dinv = jax.lax.rsqrt(colsum_ref[...] + 2.0)
    xin = xin_ref[...]
    xin_blk = xin_blk_ref[...]
    if sp_ref is not None:
        xin = xin * sp_ref[...]
        xin_blk = xin_blk * sp_blk_ref[...]
    z = jax.lax.dot_general(
        xin * dinv, w_ref[...], (((1,), (1,)), ((), ())),
        preferred_element_type=jnp.float32,
        precision=_PREC)
    y = jax.lax.dot_general(
        ap_ref[...], z, (((0,), (0,)), ((), ())),
        preferred_element_type=jnp.float32,
        precision=_PREC)
    dblk = jax.lax.rsqrt(cs_blk_ref[...] + 2.0)
    zblk = jax.lax.dot_general(
        xin_blk * dblk, w_ref[...], (((1,), (1,)), ((), ())),
        preferred_element_type=jnp.float32,
        precision=_PREC)
    g = dblk * (y + 2.0 * zblk) + b_ref[...]
    g = jnp.maximum(g, 0.0)
    mu = jnp.mean(g, axis=1, keepdims=True)
    var = jnp.mean((g - mu) ** 2, axis=1, keepdims=True)
    h = (g - mu) * jax.lax.rsqrt(var + 1e-5) * lng_ref[...] + lnb_ref[...]
    if with_time:
        tv = jax.lax.dot_general(
            t_ref[...], tw_ref[...], (((1,), (1,)), ((), ())),
            preferred_element_type=jnp.float32,
            precision=_PREC)
        h = h + jnp.maximum(tv + tb_ref[...], 0.0)
    out_ref[...] = h


# speedup vs baseline: 1.2228x; 1.0002x over previous
import dataclasses
import functools
import math

import jax
import jax.numpy as jnp
from jax import lax
from jax.experimental import pallas as pl
from jax.experimental.pallas import tpu as pltpu
from jax.experimental.pallas import tpu_sc as plsc

N = 4096
C = 256
TDIM = 512
E = 131072
K = 2048

_PREC = jax.lax.Precision.DEFAULT

E_AUG = E + K
EPT = E_AUG // 16
NB = EPT // 128
CH = 256
NCH = K // CH
ZW = 4096
GB = 16


def _build_kernel(src_hbm, dst_hbm, w_hbm, rank_hbm, perm_hbm, x_hbm,
                  p_hbm, qt_hbm, xg_hbm,
                  a_v, c_v, w_v, rank_v, idx3, r3, upd3, zeros_v,
                  permv, rows_v, shared, sem):
    cid = lax.axis_index("c")
    sid = lax.axis_index("s")
    base = sid * EPT

    wid = sid * 2 + cid
    pltpu.sync_copy(perm_hbm.at[pl.ds(wid * (K // 32), K // 32)], permv)

    @pl.loop(0, (K // 32) // GB)
    def _(g):
        pltpu.async_copy(x_hbm.at[permv.at[pl.ds(g * GB, GB)]], rows_v,
                         sem).wait()
        pltpu.sync_copy(rows_v, xg_hbm.at[pl.ds(wid * (K // 32) + g * GB, GB)])

    @pl.when(cid == 0)
    def _():
        pltpu.sync_copy(src_hbm.at[pl.ds(base, EPT)], a_v)
        pltpu.sync_copy(dst_hbm.at[pl.ds(base, EPT)], c_v)

    @pl.when(cid != 0)
    def _():
        pltpu.sync_copy(dst_hbm.at[pl.ds(base, EPT)], a_v)
        pltpu.sync_copy(src_hbm.at[pl.ds(base, EPT)], c_v)

    pltpu.sync_copy(w_hbm.at[pl.ds(base, EPT)], w_v)
    pltpu.sync_copy(rank_hbm, rank_v)

    @pl.loop(0, ZW // 16)
    def _(q):
        zeros_v[pl.ds(q * 16, 16)] = jnp.zeros((16,), jnp.float32)

    @pl.loop(0, NB)
    def _(j):
        @pl.loop(0, 8)
        def _(t):
            a16 = a_v[pl.ds(j * 128 + t * 16, 16)]
            c16 = c_v[pl.ds(j * 128 + t * 16, 16)]
            r16 = plsc.load_gather(rank_v, [a16])
            idx3[j, pl.ds(t * 16, 16)] = (r16 & (CH - 1)) * N + c16
            r3[j, pl.ds(t * 16, 16)] = r16

    part = CH * N // 16

    @pl.loop(0, NCH)
    def _(k):
        @pl.loop(0, part // ZW)
        def _(q):
            pltpu.async_copy(zeros_v, shared.at[pl.ds(sid * part + q * ZW, ZW)],
                             sem)

        @pl.loop(0, part // ZW)
        def _(q):
            pltpu.make_async_copy(
                zeros_v, shared.at[pl.ds(sid * part + q * ZW, ZW)], sem).wait()
        plsc.subcore_barrier()
        lo = k * CH

        @pl.loop(0, NB)
        def _(j):
            @pl.loop(0, 8)
            def _(t):
                r16 = r3[j, pl.ds(t * 16, 16)]
                w16 = w_v[pl.ds(j * 128 + t * 16, 16)]
                m = jnp.logical_and(r16 >= lo, r16 < lo + CH)
                upd3[j, pl.ds(t * 16, 16)] = jnp.where(m, w16, 0.0)

        @pl.loop(0, NB)
        def _(j):
            pltpu.async_copy(upd3.at[j], shared.at[idx3.at[j]], sem, add=True)

        @pl.loop(0, NB)
        def _(j):
            pltpu.make_async_copy(upd3.at[j], shared.at[idx3.at[j]], sem).wait()

        plsc.subcore_barrier()

        rows_per_tile = CH // 16

        @pl.when(cid == 0)
        def _():
            @pl.loop(0, rows_per_tile)
            def _(r):
                rl = sid * rows_per_tile + r
                pltpu.async_copy(shared.at[pl.ds(rl * N, N)],
                                 p_hbm.at[k * CH + rl], sem)

            @pl.loop(0, rows_per_tile)
            def _(r):
                rl = sid * rows_per_tile + r
                pltpu.make_async_copy(shared.at[pl.ds(rl * N, N)],
                                      p_hbm.at[k * CH + rl], sem).wait()

        @pl.when(cid != 0)
        def _():
            @pl.loop(0, rows_per_tile)
            def _(r):
                rl = sid * rows_per_tile + r
                pltpu.async_copy(shared.at[pl.ds(rl * N, N)],
                                 qt_hbm.at[k * CH + rl], sem)

            @pl.loop(0, rows_per_tile)
            def _(r):
                rl = sid * rows_per_tile + r
                pltpu.make_async_copy(shared.at[pl.ds(rl * N, N)],
                                      qt_hbm.at[k * CH + rl], sem).wait()


def _build_pq(src_aug, dst_aug, w_aug, rank_flat, perm, x):
    mesh = plsc.VectorSubcoreMesh(core_axis_name="c", subcore_axis_name="s")
    cp = pltpu.CompilerParams()
    if "needs_layout_passes" in pltpu.CompilerParams.__dataclass_fields__:
        cp = dataclasses.replace(cp, needs_layout_passes=False)
    f = pl.kernel(
        _build_kernel,
        compiler_params=cp,
        out_type=[jax.ShapeDtypeStruct((K, N), jnp.float32),
                  jax.ShapeDtypeStruct((K, N), jnp.float32),
                  jax.ShapeDtypeStruct((K, C), jnp.float32)],
        mesh=mesh,
        scratch_types=[
            pltpu.VMEM((EPT,), jnp.int32),
            pltpu.VMEM((EPT,), jnp.int32),
            pltpu.VMEM((EPT,), jnp.float32),
            pltpu.VMEM((N,), jnp.int32),
            pltpu.VMEM((NB, 128), jnp.int32),
            pltpu.VMEM((NB, 128), jnp.int32),
            pltpu.VMEM((NB, 128), jnp.float32),
            pltpu.VMEM((ZW,), jnp.float32),
            pltpu.VMEM((K // 32,), jnp.int32),
            pltpu.VMEM((GB, C), jnp.float32),
            pltpu.VMEM_SHARED((CH * N,), jnp.float32),
            pltpu.SemaphoreType.DMA,
        ])
    return f(src_aug, dst_aug, w_aug, rank_flat, perm, x)


def _rank_kernel(scol_ref, srow_ref, rank_ref):
    i = pl.program_id(0)
    bm = scol_ref.shape[0]
    sc = scol_ref[...]
    sr = srow_ref[...]
    gt = (sr > sc).astype(jnp.int32)
    jrow = jax.lax.broadcasted_iota(jnp.int32, (bm, N), 1)
    gidx = i * bm + jax.lax.broadcasted_iota(jnp.int32, (bm, N), 0)
    tie = jnp.logical_and(sr == sc, jrow < gidx).astype(jnp.int32)
    rank_ref[...] = jnp.sum(gt + tie, axis=1, keepdims=True)


def _perm_kernel(rrow_ref, srow_ref, perm_ref, sp_ref):
    i = pl.program_id(0)
    bm = perm_ref.shape[0]
    rr = rrow_ref[...]
    sr = srow_ref[...]
    p = i * bm + jax.lax.broadcasted_iota(jnp.int32, (bm, N), 0)
    hit = (rr == p)
    jrow = jax.lax.broadcasted_iota(jnp.int32, (bm, N), 1)
    perm_ref[...] = jnp.sum(jnp.where(hit, jrow, 0), axis=1, keepdims=True)
    sp_ref[...] = jnp.sum(jnp.where(hit, sr, 0.0), axis=1, keepdims=True)


def _ap_kernel(p_ref, qt_ref, ap_ref, apb_ref, colsum_ref, *, bm, bn):
    j = pl.program_id(0)
    i = pl.program_id(1)
    acc = jax.lax.dot_general(
        p_ref[...], qt_ref[...], (((1,), (1,)), ((), ())),
        preferred_element_type=jnp.float32,
        precision=_PREC)
    gi = i * bm + jax.lax.broadcasted_iota(jnp.int32, (bm, bn), 0)
    gj = j * bn + jax.lax.broadcasted_iota(jnp.int32, (bm, bn), 1)
    acc = jnp.where(gi == gj, 0.0, acc)
    ap_ref[...] = acc
    apb_ref[...] = acc.astype(jnp.bfloat16)
    part = jnp.sum(acc, axis=0, keepdims=True)[None]

    @pl.when(i == 0)
    def _init():
        colsum_ref[...] = part

    @pl.when(i != 0)
    def _acc():
        colsum_ref[...] += part


def kernel(x, edge_index, edge_weight, t, W1, b1, W2, b2,
           ln1_g, ln1_b, ln2_g, ln2_b, pool_w, time_W, time_b):
    src, dst = edge_index[0], edge_index[1]

    score = (x * pool_w).sum(-1) / jnp.linalg.norm(pool_w)
    score = jnp.tanh(score)

    srow = score.reshape(1, N)
    scol = score.reshape(N, 1)

    bm = 512
    rank = pl.pallas_call(
        _rank_kernel,
        grid=(N // bm,),
        in_specs=[pl.BlockSpec((bm, 1), lambda i: (i, 0)),
                  pl.BlockSpec((1, N), lambda i: (0, 0))],
        out_specs=pl.BlockSpec((bm, 1), lambda i: (i, 0)),
        out_shape=jax.ShapeDtypeStruct((N, 1), jnp.int32),
    )(scol, srow)

    rrow = rank.reshape(1, N)
    perm_full, sp_full = pl.pallas_call(
        _perm_kernel,
        grid=(N // bm,),
        in_specs=[pl.BlockSpec((1, N), lambda i: (0, 0)),
                  pl.BlockSpec((1, N), lambda i: (0, 0))],
        out_specs=[pl.BlockSpec((bm, 1), lambda i: (i, 0)),
                   pl.BlockSpec((bm, 1), lambda i: (i, 0))],
        out_shape=[jax.ShapeDtypeStruct((N, 1), jnp.int32),
                   jax.ShapeDtypeStruct((N, 1), jnp.float32)],
    )(rrow, srow)
    perm = perm_full[:K, 0]
    sp = sp_full[:K]

    rk = rank[:, 0]
    w0 = jnp.where(src == dst, 0.0, edge_weight)
    src_aug = jnp.concatenate([src, perm])
    dst_aug = jnp.concatenate([dst, perm])
    w_aug = jnp.concatenate([w0, jnp.ones((K,), jnp.float32)])
    P, QT, xg = _build_pq(src_aug, dst_aug, w_aug, rk, perm, x)

    bm3, bn3 = 256, 1024
    Pb = P.astype(jnp.bfloat16)
    QTb = QT.astype(jnp.bfloat16)
    ap, apb, colsum = pl.pallas_call(
        functools.partial(_ap_kernel, bm=bm3, bn=bn3),
        grid=(K // bn3, K // bm3),
        in_specs=[pl.BlockSpec((bm3, N), lambda j, i: (i, 0)),
                  pl.BlockSpec((bn3, N), lambda j, i: (j, 0))],
        out_specs=[pl.BlockSpec((bm3, bn3), lambda j, i: (i, j)),
                   pl.BlockSpec((bm3, bn3), lambda j, i: (i, j)),
                   pl.BlockSpec((1, 1, bn3), lambda j, i: (j, 0, 0))],
        out_shape=[jax.ShapeDtypeStruct((K, K), jnp.float32),
                   jax.ShapeDtypeStruct((K, K), jnp.bfloat16),
                   jax.ShapeDtypeStruct((K // bn3, 1, bn3), jnp.float32)],
        compiler_params=pltpu.CompilerParams(
            dimension_semantics=("parallel", "arbitrary")),
    )(Pb, QTb)
    colsum = colsum.reshape(K, 1)

    bm4 = 512
    t_row = t.reshape(1, TDIM)
    tb_row = time_b.reshape(1, C)

    def gcn_layer(xin, spv, W, b, lng, lnb, with_time):
        body = functools.partial(_gcn2_kernel, bm=bm4, with_time=with_time,
                                 with_sp=spv is not None)
        ins = [apb, colsum, xin]
        specs = [pl.BlockSpec((K, bm4), lambda i: (0, i)),
                 pl.BlockSpec((K, 1), lambda i: (0, 0)),
                 pl.BlockSpec((K, C), lambda i: (0, 0))]
        if spv is not None:
            ins.append(spv)
            specs.append(pl.BlockSpec((K, 1), lambda i: (0, 0)))
        ins += [colsum, xin]
        specs += [pl.BlockSpec((bm4, 1), lambda i: (i, 0)),
                  pl.BlockSpec((bm4, C), lambda i: (i, 0))]
        if spv is not None:
            ins.append(spv)
            specs.append(pl.BlockSpec((bm4, 1), lambda i: (i, 0)))
        ins += [W, b.reshape(1, C), lng.reshape(1, C), lnb.reshape(1, C)]
        specs += [pl.BlockSpec((C, C), lambda i: (0, 0)),
                  pl.BlockSpec((1, C), lambda i: (0, 0)),
                  pl.BlockSpec((1, C), lambda i: (0, 0)),
                  pl.BlockSpec((1, C), lambda i: (0, 0))]
        if with_time:
            ins += [time_W, t_row, tb_row]
            specs += [pl.BlockSpec((C, TDIM), lambda i: (0, 0)),
                      pl.BlockSpec((1, TDIM), lambda i: (0, 0)),
                      pl.BlockSpec((1, C), lambda i: (0, 0))]
        return pl.pallas_call(
            body,
            grid=(K // bm4,),
            in_specs=specs,
            out_specs=pl.BlockSpec((bm4, C), lambda i: (i, 0)),
            out_shape=jax.ShapeDtypeStruct((K, C), jnp.float32),
            compiler_params=pltpu.CompilerParams(
                dimension_semantics=("parallel",)),
            )(*ins)

    h1 = gcn_layer(xg, sp, W1, b1, ln1_g, ln1_b, True)
    h = gcn_layer(h1, None, W2, b2, ln2_g, ln2_b, False)
    return h, ap, perm


def _gcn2_kernel(*refs, bm, with_time, with_sp):
    if with_time:
        *refs, tw_ref, t_ref, tb_ref, out_ref = refs
        refs = refs + [out_ref]
    if with_sp:
        (ap_ref, colsum_ref, xin_ref, sp_ref, cs_blk_ref, xin_blk_ref,
         sp_blk_ref, w_ref, b_ref, lng_ref, lnb_ref, out_ref) = refs
    else:
        (ap_ref, colsum_ref, xin_ref, cs_blk_ref, xin_blk_ref,
         w_ref, b_ref, lng_ref, lnb_ref, out_ref) = refs
        sp_ref = sp_blk_ref = None
    dinv = jax.lax.rsqrt(colsum_ref[...] + 2.0)
    xin = xin_ref[...]
    xin_blk = xin_blk_ref[...]
    if sp_ref is not None:
        xin = xin * sp_ref[...]
        xin_blk = xin_blk * sp_blk_ref[...]
    z = jax.lax.dot_general(
        xin * dinv, w_ref[...], (((1,), (1,)), ((), ())),
        preferred_element_type=jnp.float32,
        precision=_PREC)
    y = jax.lax.dot_general(
        ap_ref[...], z.astype(jnp.bfloat16), (((0,), (0,)), ((), ())),
        preferred_element_type=jnp.float32,
        precision=_PREC)
    dblk = jax.lax.rsqrt(cs_blk_ref[...] + 2.0)
    zblk = jax.lax.dot_general(
        xin_blk * dblk, w_ref[...], (((1,), (1,)), ((), ())),
        preferred_element_type=jnp.float32,
        precision=_PREC)
    g = dblk * (y + 2.0 * zblk) + b_ref[...]
    g = jnp.maximum(g, 0.0)
    mu = jnp.mean(g, axis=1, keepdims=True)
    var = jnp.mean((g - mu) ** 2, axis=1, keepdims=True)
    h = (g - mu) * jax.lax.rsqrt(var + 1e-5) * lng_ref[...] + lnb_ref[...]
    if with_time:
        tv = jax.lax.dot_general(
            t_ref[...], tw_ref[...], (((1,), (1,)), ((), ())),
            preferred_element_type=jnp.float32,
            precision=_PREC)
        h = h + jnp.maximum(tv + tb_ref[...], 0.0)
    out_ref[...] = h


# speedup vs baseline: 1.2705x; 1.0390x over previous
import dataclasses
import functools
import math

import jax
import jax.numpy as jnp
from jax import lax
from jax.experimental import pallas as pl
from jax.experimental.pallas import tpu as pltpu
from jax.experimental.pallas import tpu_sc as plsc

N = 4096
C = 256
TDIM = 512
E = 131072
K = 2048

_PREC = jax.lax.Precision.DEFAULT

E_AUG = E + K
EPT = E_AUG // 16
NB = EPT // 128
CH = 256
NCH = K // CH
ZW = 4096
GB = 16


def _build_kernel(src_hbm, dst_hbm, w_hbm, rank_hbm, perm_hbm, x_hbm,
                  p_hbm, qt_hbm, xg_hbm,
                  a_v, c_v, w_v, rank_v, idx3, r3, upd3, zeros_v,
                  permv, rows_v, shared, sem):
    cid = lax.axis_index("c")
    sid = lax.axis_index("s")
    base = sid * EPT

    wid = sid * 2 + cid
    pltpu.sync_copy(perm_hbm.at[pl.ds(wid * (K // 32), K // 32)], permv)

    @pl.loop(0, (K // 32) // GB)
    def _(g):
        pltpu.async_copy(x_hbm.at[permv.at[pl.ds(g * GB, GB)]], rows_v,
                         sem).wait()
        pltpu.sync_copy(rows_v, xg_hbm.at[pl.ds(wid * (K // 32) + g * GB, GB)])

    @pl.when(cid == 0)
    def _():
        pltpu.sync_copy(src_hbm.at[pl.ds(base, EPT)], a_v)
        pltpu.sync_copy(dst_hbm.at[pl.ds(base, EPT)], c_v)

    @pl.when(cid != 0)
    def _():
        pltpu.sync_copy(dst_hbm.at[pl.ds(base, EPT)], a_v)
        pltpu.sync_copy(src_hbm.at[pl.ds(base, EPT)], c_v)

    pltpu.sync_copy(w_hbm.at[pl.ds(base, EPT)], w_v)
    pltpu.sync_copy(rank_hbm, rank_v)

    @pl.loop(0, ZW // 16)
    def _(q):
        zeros_v[pl.ds(q * 16, 16)] = jnp.zeros((16,), jnp.float32)

    @pl.loop(0, NB)
    def _(j):
        @pl.loop(0, 8)
        def _(t):
            a16 = a_v[pl.ds(j * 128 + t * 16, 16)]
            c16 = c_v[pl.ds(j * 128 + t * 16, 16)]
            r16 = plsc.load_gather(rank_v, [a16])
            idx3[j, pl.ds(t * 16, 16)] = (r16 & (CH - 1)) * N + c16
            r3[j, pl.ds(t * 16, 16)] = r16

    part = CH * N // 16

    @pl.loop(0, NCH)
    def _(k):
        @pl.loop(0, part // ZW)
        def _(q):
            pltpu.async_copy(zeros_v, shared.at[pl.ds(sid * part + q * ZW, ZW)],
                             sem)

        @pl.loop(0, part // ZW)
        def _(q):
            pltpu.make_async_copy(
                zeros_v, shared.at[pl.ds(sid * part + q * ZW, ZW)], sem).wait()
        plsc.subcore_barrier()
        lo = k * CH

        @pl.loop(0, NB)
        def _(j):
            @pl.loop(0, 8)
            def _(t):
                r16 = r3[j, pl.ds(t * 16, 16)]
                w16 = w_v[pl.ds(j * 128 + t * 16, 16)]
                m = jnp.logical_and(r16 >= lo, r16 < lo + CH)
                upd3[j, pl.ds(t * 16, 16)] = jnp.where(m, w16, 0.0)

        @pl.loop(0, NB)
        def _(j):
            pltpu.async_copy(upd3.at[j], shared.at[idx3.at[j]], sem, add=True)

        @pl.loop(0, NB)
        def _(j):
            pltpu.make_async_copy(upd3.at[j], shared.at[idx3.at[j]], sem).wait()

        plsc.subcore_barrier()

        rows_per_tile = CH // 16

        @pl.when(cid == 0)
        def _():
            @pl.loop(0, rows_per_tile)
            def _(r):
                rl = sid * rows_per_tile + r
                pltpu.async_copy(shared.at[pl.ds(rl * N, N)],
                                 p_hbm.at[k * CH + rl], sem)

            @pl.loop(0, rows_per_tile)
            def _(r):
                rl = sid * rows_per_tile + r
                pltpu.make_async_copy(shared.at[pl.ds(rl * N, N)],
                                      p_hbm.at[k * CH + rl], sem).wait()

        @pl.when(cid != 0)
        def _():
            @pl.loop(0, rows_per_tile)
            def _(r):
                rl = sid * rows_per_tile + r
                pltpu.async_copy(shared.at[pl.ds(rl * N, N)],
                                 qt_hbm.at[k * CH + rl], sem)

            @pl.loop(0, rows_per_tile)
            def _(r):
                rl = sid * rows_per_tile + r
                pltpu.make_async_copy(shared.at[pl.ds(rl * N, N)],
                                      qt_hbm.at[k * CH + rl], sem).wait()


def _build_pq(src_aug, dst_aug, w_aug, rank_flat, perm, x):
    mesh = plsc.VectorSubcoreMesh(core_axis_name="c", subcore_axis_name="s")
    cp = pltpu.CompilerParams()
    if "needs_layout_passes" in pltpu.CompilerParams.__dataclass_fields__:
        cp = dataclasses.replace(cp, needs_layout_passes=False)
    f = pl.kernel(
        _build_kernel,
        compiler_params=cp,
        out_type=[jax.ShapeDtypeStruct((K, N), jnp.float32),
                  jax.ShapeDtypeStruct((K, N), jnp.float32),
                  jax.ShapeDtypeStruct((K, C), jnp.float32)],
        mesh=mesh,
        scratch_types=[
            pltpu.VMEM((EPT,), jnp.int32),
            pltpu.VMEM((EPT,), jnp.int32),
            pltpu.VMEM((EPT,), jnp.float32),
            pltpu.VMEM((N,), jnp.int32),
            pltpu.VMEM((NB, 128), jnp.int32),
            pltpu.VMEM((NB, 128), jnp.int32),
            pltpu.VMEM((NB, 128), jnp.float32),
            pltpu.VMEM((ZW,), jnp.float32),
            pltpu.VMEM((K // 32,), jnp.int32),
            pltpu.VMEM((GB, C), jnp.float32),
            pltpu.VMEM_SHARED((CH * N,), jnp.float32),
            pltpu.SemaphoreType.DMA,
        ])
    return f(src_aug, dst_aug, w_aug, rank_flat, perm, x)


def _rank_kernel(scol_ref, srow_ref, rank_ref):
    i = pl.program_id(0)
    bm = scol_ref.shape[0]
    sc = scol_ref[...]
    sr = srow_ref[...]
    gt = (sr > sc).astype(jnp.int32)
    jrow = jax.lax.broadcasted_iota(jnp.int32, (bm, N), 1)
    gidx = i * bm + jax.lax.broadcasted_iota(jnp.int32, (bm, N), 0)
    tie = jnp.logical_and(sr == sc, jrow < gidx).astype(jnp.int32)
    rank_ref[...] = jnp.sum(gt + tie, axis=1, keepdims=True)


def _perm_kernel(rrow_ref, srow_ref, perm_ref, sp_ref):
    i = pl.program_id(0)
    bm = perm_ref.shape[0]
    rr = rrow_ref[...]
    sr = srow_ref[...]
    p = i * bm + jax.lax.broadcasted_iota(jnp.int32, (bm, N), 0)
    hit = (rr == p)
    jrow = jax.lax.broadcasted_iota(jnp.int32, (bm, N), 1)
    perm_ref[...] = jnp.sum(jnp.where(hit, jrow, 0), axis=1, keepdims=True)
    sp_ref[...] = jnp.sum(jnp.where(hit, sr, 0.0), axis=1, keepdims=True)


def _ap_kernel(p_ref, qt_ref, ap_ref, apb_ref, colsum_ref, *, bm, bn):
    j = pl.program_id(0)
    i = pl.program_id(1)
    acc = jax.lax.dot_general(
        p_ref[...].astype(jnp.bfloat16), qt_ref[...], (((1,), (1,)), ((), ())),
        preferred_element_type=jnp.float32,
        precision=_PREC)
    gi = i * bm + jax.lax.broadcasted_iota(jnp.int32, (bm, bn), 0)
    gj = j * bn + jax.lax.broadcasted_iota(jnp.int32, (bm, bn), 1)
    acc = jnp.where(gi == gj, 0.0, acc)
    ap_ref[...] = acc
    apb_ref[...] = acc.astype(jnp.bfloat16)
    part = jnp.sum(acc, axis=0, keepdims=True)[None]

    @pl.when(i == 0)
    def _init():
        colsum_ref[...] = part

    @pl.when(i != 0)
    def _acc():
        colsum_ref[...] += part


def kernel(x, edge_index, edge_weight, t, W1, b1, W2, b2,
           ln1_g, ln1_b, ln2_g, ln2_b, pool_w, time_W, time_b):
    src, dst = edge_index[0], edge_index[1]

    score = (x * pool_w).sum(-1) / jnp.linalg.norm(pool_w)
    score = jnp.tanh(score)

    srow = score.reshape(1, N)
    scol = score.reshape(N, 1)

    bm = 512
    rank = pl.pallas_call(
        _rank_kernel,
        grid=(N // bm,),
        in_specs=[pl.BlockSpec((bm, 1), lambda i: (i, 0)),
                  pl.BlockSpec((1, N), lambda i: (0, 0))],
        out_specs=pl.BlockSpec((bm, 1), lambda i: (i, 0)),
        out_shape=jax.ShapeDtypeStruct((N, 1), jnp.int32),
    )(scol, srow)

    rrow = rank.reshape(1, N)
    perm_full, sp_full = pl.pallas_call(
        _perm_kernel,
        grid=(N // bm,),
        in_specs=[pl.BlockSpec((1, N), lambda i: (0, 0)),
                  pl.BlockSpec((1, N), lambda i: (0, 0))],
        out_specs=[pl.BlockSpec((bm, 1), lambda i: (i, 0)),
                   pl.BlockSpec((bm, 1), lambda i: (i, 0))],
        out_shape=[jax.ShapeDtypeStruct((N, 1), jnp.int32),
                   jax.ShapeDtypeStruct((N, 1), jnp.float32)],
    )(rrow, srow)
    perm = perm_full[:K, 0]
    sp = sp_full[:K]

    rk = rank[:, 0]
    w0 = jnp.where(src == dst, 0.0, edge_weight)
    src_aug = jnp.concatenate([src, perm])
    dst_aug = jnp.concatenate([dst, perm])
    w_aug = jnp.concatenate([w0, jnp.ones((K,), jnp.float32)])
    P, QT, xg = _build_pq(src_aug, dst_aug, w_aug, rk, perm, x)

    bm3, bn3 = 256, 1024
    QTb = QT.astype(jnp.bfloat16)
    ap, apb, colsum = pl.pallas_call(
        functools.partial(_ap_kernel, bm=bm3, bn=bn3),
        grid=(K // bn3, K // bm3),
        in_specs=[pl.BlockSpec((bm3, N), lambda j, i: (i, 0)),
                  pl.BlockSpec((bn3, N), lambda j, i: (j, 0))],
        out_specs=[pl.BlockSpec((bm3, bn3), lambda j, i: (i, j)),
                   pl.BlockSpec((bm3, bn3), lambda j, i: (i, j)),
                   pl.BlockSpec((1, 1, bn3), lambda j, i: (j, 0, 0))],
        out_shape=[jax.ShapeDtypeStruct((K, K), jnp.float32),
                   jax.ShapeDtypeStruct((K, K), jnp.bfloat16),
                   jax.ShapeDtypeStruct((K // bn3, 1, bn3), jnp.float32)],
        compiler_params=pltpu.CompilerParams(
            dimension_semantics=("parallel", "arbitrary")),
    )(P, QTb)
    colsum = colsum.reshape(K, 1)

    bm4 = 512
    t_row = t.reshape(1, TDIM)
    tb_row = time_b.reshape(1, C)

    def gcn_layer(xin, spv, W, b, lng, lnb, with_time):
        body = functools.partial(_gcn2_kernel, bm=bm4, with_time=with_time,
                                 with_sp=spv is not None)
        ins = [apb, colsum, xin]
        specs = [pl.BlockSpec((K, bm4), lambda i: (0, i)),
                 pl.BlockSpec((K, 1), lambda i: (0, 0)),
                 pl.BlockSpec((K, C), lambda i: (0, 0))]
        if spv is not None:
            ins.append(spv)
            specs.append(pl.BlockSpec((K, 1), lambda i: (0, 0)))
        ins += [colsum, xin]
        specs += [pl.BlockSpec((bm4, 1), lambda i: (i, 0)),
                  pl.BlockSpec((bm4, C), lambda i: (i, 0))]
        if spv is not None:
            ins.append(spv)
            specs.append(pl.BlockSpec((bm4, 1), lambda i: (i, 0)))
        ins += [W, b.reshape(1, C), lng.reshape(1, C), lnb.reshape(1, C)]
        specs += [pl.BlockSpec((C, C), lambda i: (0, 0)),
                  pl.BlockSpec((1, C), lambda i: (0, 0)),
                  pl.BlockSpec((1, C), lambda i: (0, 0)),
                  pl.BlockSpec((1, C), lambda i: (0, 0))]
        if with_time:
            ins += [time_W, t_row, tb_row]
            specs += [pl.BlockSpec((C, TDIM), lambda i: (0, 0)),
                      pl.BlockSpec((1, TDIM), lambda i: (0, 0)),
                      pl.BlockSpec((1, C), lambda i: (0, 0))]
        return pl.pallas_call(
            body,
            grid=(K // bm4,),
            in_specs=specs,
            out_specs=pl.BlockSpec((bm4, C), lambda i: (i, 0)),
            out_shape=jax.ShapeDtypeStruct((K, C), jnp.float32),
            compiler_params=pltpu.CompilerParams(
                dimension_semantics=("parallel",)),
            )(*ins)

    h1 = gcn_layer(xg, sp, W1, b1, ln1_g, ln1_b, True)
    h = gcn_layer(h1, None, W2, b2, ln2_g, ln2_b, False)
    return h, ap, perm


def _gcn2_kernel(*refs, bm, with_time, with_sp):
    if with_time:
        *refs, tw_ref, t_ref, tb_ref, out_ref = refs
        refs = refs + [out_ref]
    if with_sp:
        (ap_ref, colsum_ref, xin_ref, sp_ref, cs_blk_ref, xin_blk_ref,
         sp_blk_ref, w_ref, b_ref, lng_ref, lnb_ref, out_ref) = refs
    else:
        (ap_ref, colsum_ref, xin_ref, cs_blk_ref, xin_blk_ref,
         w_ref, b_ref, lng_ref, lnb_ref, out_ref) = refs
        sp_ref = sp_blk_ref = None
    dinv = jax.lax.rsqrt(colsum_ref[...] + 2.0)
    xin = xin_ref[...]
    xin_blk = xin_blk_ref[...]
    if sp_ref is not None:
        xin = xin * sp_ref[...]
        xin_blk = xin_blk * sp_blk_ref[...]
    z = jax.lax.dot_general(
        xin * dinv, w_ref[...], (((1,), (1,)), ((), ())),
        preferred_element_type=jnp.float32,
        precision=_PREC)
    y = jax.lax.dot_general(
        ap_ref[...], z.astype(jnp.bfloat16), (((0,), (0,)), ((), ())),
        preferred_element_type=jnp.float32,
        precision=_PREC)
    dblk = jax.lax.rsqrt(cs_blk_ref[...] + 2.0)
    zblk = jax.lax.dot_general(
        xin_blk * dblk, w_ref[...], (((1,), (1,)), ((), ())),
        preferred_element_type=jnp.float32,
        precision=_PREC)
    g = dblk * (y + 2.0 * zblk) + b_ref[...]
    g = jnp.maximum(g, 0.0)
    mu = jnp.mean(g, axis=1, keepdims=True)
    var = jnp.mean((g - mu) ** 2, axis=1, keepdims=True)
    h = (g - mu) * jax.lax.rsqrt(var + 1e-5) * lng_ref[...] + lnb_ref[...]
    if with_time:
        tv = jax.lax.dot_general(
            t_ref[...], tw_ref[...], (((1,), (1,)), ((), ())),
            preferred_element_type=jnp.float32,
            precision=_PREC)
        h = h + jnp.maximum(tv + tb_ref[...], 0.0)
    out_ref[...] = h
